# baseline (speedup 1.0000x reference)
import functools

import jax
import jax.numpy as jnp
from jax import lax
from jax.experimental import pallas as pl
from jax.experimental.pallas import tpu as pltpu
from jax.experimental.pallas import tpu_sc as plsc

N = 10000
E = 320000
T = 8
A = 16
H = 128
I = 64

NTILES = 32
LANES = 16
CHUNK = 64
GCH = 16
NG = 10
CH = NG * GCH
RPT_E = E // NTILES
EPT = CH * CHUNK
TPAD = EPT - RPT_E
RPT = 640
NP = 16 * RPT
NB = 640
GRIDA = NP // NB
NB2 = 1024
GRID2 = NP // NB2

_mesh = plsc.VectorSubcoreMesh(core_axis_name="c", subcore_axis_name="s")
_sc_params = pltpu.CompilerParams(needs_layout_passes=False,
                                  use_tc_tiling_on_sc=False)


@functools.partial(
    pl.kernel,
    out_type=(jax.ShapeDtypeStruct((NTILES, NP), jnp.float32),
              jax.ShapeDtypeStruct((NTILES, NP), jnp.float32)),
    mesh=_mesh,
    scratch_types=[
        pltpu.VMEM((RPT_E,), jnp.int32),
        pltpu.VMEM((RPT_E,), jnp.int32),
        pltpu.VMEM((NP,), jnp.float32),
        pltpu.VMEM((NP,), jnp.float32),
    ],
    compiler_params=_sc_params,
)
def _sc_degrees(edge_hbm, dego_hbm, degi_hbm, src_v, dst_v, ho_v, hi_v):
    c = lax.axis_index("c")
    s = lax.axis_index("s")
    wid = c * 16 + s
    pltpu.sync_copy(edge_hbm.at[0, pl.ds(wid * RPT_E, RPT_E)], src_v)
    pltpu.sync_copy(edge_hbm.at[1, pl.ds(wid * RPT_E, RPT_E)], dst_v)
    zeros = jnp.zeros((LANES,), jnp.float32)
    ones = jnp.ones((LANES,), jnp.float32)

    def zero_body(v, _):
        ho_v[pl.ds(v * LANES, LANES)] = zeros
        hi_v[pl.ds(v * LANES, LANES)] = zeros
        return _

    lax.fori_loop(0, NP // LANES, zero_body, 0)

    def hist_body(v, _):
        si = src_v[pl.ds(v * LANES, LANES)]
        di = dst_v[pl.ds(v * LANES, LANES)]
        plsc.addupdate_scatter(ho_v, [si], ones)
        plsc.addupdate_scatter(hi_v, [di], ones)
        return _

    lax.fori_loop(0, RPT_E // LANES, hist_body, 0)
    pltpu.sync_copy(ho_v, dego_hbm.at[wid])
    pltpu.sync_copy(hi_v, degi_hbm.at[wid])


def _tc_gru_body(a_ref, h0_ref, wih_ref, whh_ref, bih_ref, bhh_ref,
                 lng_ref, lnb_ref, hfin_ref, xln_ref):
    blk = pl.program_id(0)
    wih = wih_ref[...]
    whh = whh_ref[...]
    bih = bih_ref[...]
    bhh = bhh_ref[...]
    a = a_ref[...]
    h = h0_ref[0]
    for t in range(T):
        x_t = a[:, t * A:(t + 1) * A]
        gi = jnp.dot(x_t, wih, preferred_element_type=jnp.float32) + bih
        gh = jnp.dot(h, whh, preferred_element_type=jnp.float32) + bhh
        i_r = gi[:, :H]
        i_z = gi[:, H:2 * H]
        i_n = gi[:, 2 * H:]
        h_r = gh[:, :H]
        h_z = gh[:, H:2 * H]
        h_n = gh[:, 2 * H:]
        r = jax.nn.sigmoid(i_r + h_r)
        z = jax.nn.sigmoid(i_z + h_z)
        n = jnp.tanh(i_n + r * h_n)
        h = (1.0 - z) * n + z * h
    hfin_ref[...] = h

    mu = jnp.mean(h, axis=1, keepdims=True)
    d = h - mu
    var = jnp.mean(d * d, axis=1, keepdims=True)
    x = d * lax.rsqrt(var + 1e-5) * lng_ref[...] + lnb_ref[...]
    row = jax.lax.broadcasted_iota(jnp.int32, (NB, 1), 0) + blk * NB
    xln_ref[...] = jnp.where(row < N, x, 0.0)


def _tc_gru(acts, hidden, wihT, whhT, bih, bhh, lng, lnb):
    return pl.pallas_call(
        _tc_gru_body,
        grid=(GRIDA,),
        in_specs=[
            pl.BlockSpec((NB, T * A), lambda i: (i, 0)),
            pl.BlockSpec((1, NB, H), lambda i: (0, i, 0)),
            pl.BlockSpec((A, 3 * H), lambda i: (0, 0)),
            pl.BlockSpec((H, 3 * H), lambda i: (0, 0)),
            pl.BlockSpec((1, 3 * H), lambda i: (0, 0)),
            pl.BlockSpec((1, 3 * H), lambda i: (0, 0)),
            pl.BlockSpec((1, H), lambda i: (0, 0)),
            pl.BlockSpec((1, H), lambda i: (0, 0)),
        ],
        out_specs=[
            pl.BlockSpec((NB, H), lambda i: (i, 0)),
            pl.BlockSpec((NB, H), lambda i: (i, 0)),
        ],
        out_shape=[
            jax.ShapeDtypeStruct((N, H), jnp.float32),
            jax.ShapeDtypeStruct((NP, H), jnp.float32),
        ],
    )(acts, hidden, wihT, whhT, bih, bhh, lng, lnb)


def _tc_scale_body(x_ref, degp_ref, out_ref):
    deg = jnp.sum(degp_ref[...], axis=0).reshape(NB, 1)
    norm_src = lax.rsqrt(jnp.maximum(deg, 1.0))
    out_ref[...] = x_ref[...] * norm_src


def _tc_scale(x_ln, deg_out_parts):
    return pl.pallas_call(
        _tc_scale_body,
        grid=(GRIDA,),
        in_specs=[
            pl.BlockSpec((NB, H), lambda i: (i, 0)),
            pl.BlockSpec((NTILES, NB), lambda i: (0, i)),
        ],
        out_specs=pl.BlockSpec((NB, H), lambda i: (i, 0)),
        out_shape=jax.ShapeDtypeStruct((NP, H), jnp.float32),
    )(x_ln, deg_out_parts)


@functools.partial(
    pl.kernel,
    out_type=jax.ShapeDtypeStruct((2, NP, H), jnp.float32),
    mesh=_mesh,
    scratch_types=[
        pltpu.VMEM((2, GCH, CHUNK), jnp.int32),
        pltpu.VMEM((2, GCH, CHUNK), jnp.int32),
        pltpu.VMEM((4, CHUNK, H), jnp.float32),
        pltpu.VMEM_SHARED((NP, H), jnp.float32),
        pltpu.SemaphoreType.DMA,
        pltpu.SemaphoreType.DMA,
        pltpu.SemaphoreType.DMA,
        pltpu.SemaphoreType.DMA,
    ],
    compiler_params=_sc_params,
)
def _sc_gather_scatter(x_hbm, src_hbm, dst_hbm, zeros_hbm, out_hbm,
                       src_v, dst_v, bufs, agg_sh, g0, g1, s0, s1):
    c = lax.axis_index("c")
    s = lax.axis_index("s")
    wid = c * 16 + s
    gsems = (g0, g1)
    ssems = (s0, s1)

    def srow(ci):
        return src_v.at[(ci // GCH) % 2, ci % GCH]

    def drow(ci):
        return dst_v.at[(ci // GCH) % 2, ci % GCH]

    pltpu.sync_copy(zeros_hbm, agg_sh.at[pl.ds(s * RPT, RPT)])
    pltpu.sync_copy(src_hbm.at[wid, 0], src_v.at[0])
    pltpu.sync_copy(dst_hbm.at[wid, 0], dst_v.at[0])
    plsc.subcore_barrier()

    pltpu.async_copy(x_hbm.at[srow(0)], bufs.at[0], g0)
    pltpu.async_copy(x_hbm.at[srow(1)], bufs.at[1], g1)

    def group(g, _):
        slot = g % 2
        nslot = (g + 1) % 2
        for j in range(GCH):
            if j == 2:
                @pl.when(g < NG - 1)
                def _prefetch():
                    pltpu.sync_copy(src_hbm.at[wid, g + 1], src_v.at[nslot])
                    pltpu.sync_copy(dst_hbm.at[wid, g + 1], dst_v.at[nslot])
            pltpu.make_async_copy(x_hbm.at[src_v.at[slot, j]],
                                  bufs.at[j % 4], gsems[j % 2]).wait()
            if j >= 2:
                pltpu.make_async_copy(bufs.at[(j - 2) % 4],
                                      agg_sh.at[dst_v.at[slot, j - 2]],
                                      ssems[j % 2]).wait()
            else:
                @pl.when(g > 0)
                def _wait_prev():
                    pltpu.make_async_copy(bufs.at[(j - 2) % 4],
                                          agg_sh.at[dst_v.at[nslot,
                                                             GCH + j - 2]],
                                          ssems[j % 2]).wait()
            pltpu.async_copy(bufs.at[j % 4], agg_sh.at[dst_v.at[slot, j]],
                             ssems[j % 2], add=True)
            if j < GCH - 2:
                pltpu.async_copy(x_hbm.at[src_v.at[slot, j + 2]],
                                 bufs.at[(j + 2) % 4], gsems[j % 2])
            else:
                @pl.when(g < NG - 1)
                def _fire_next():
                    pltpu.async_copy(x_hbm.at[src_v.at[nslot, j + 2 - GCH]],
                                     bufs.at[(j + 2) % 4], gsems[j % 2])
        return _

    lax.fori_loop(0, NG, group, 0)

    pltpu.make_async_copy(bufs.at[(CH - 2) % 4], agg_sh.at[drow(CH - 2)],
                          ssems[(CH - 2) % 2]).wait()
    pltpu.make_async_copy(bufs.at[(CH - 1) % 4], agg_sh.at[drow(CH - 1)],
                          ssems[(CH - 1) % 2]).wait()

    plsc.subcore_barrier()
    pltpu.sync_copy(agg_sh.at[pl.ds(s * RPT, RPT)],
                    out_hbm.at[c, pl.ds(s * RPT, RPT)])


def _tc_heads_body(agg2_ref, degp_ref, gw_ref, gb_ref,
                   wam_ref, bam_ref, was_ref, bas_ref,
                   wgm_ref, bgm_ref, wgs_ref, bgs_ref, out_ref):
    agg = agg2_ref[0] + agg2_ref[1]
    deg = jnp.sum(degp_ref[...], axis=0).reshape(NB2, 1)
    norm_dst = lax.rsqrt(jnp.maximum(deg, 1.0))
    hg = jnp.dot(agg * norm_dst, gw_ref[...],
                 preferred_element_type=jnp.float32) + gb_ref[...]
    out_ref[0] = jnp.dot(hg, wam_ref[...],
                         preferred_element_type=jnp.float32) + bam_ref[...]
    out_ref[1] = jax.nn.softplus(
        jnp.dot(hg, was_ref[...], preferred_element_type=jnp.float32)
        + bas_ref[...])
    out_ref[2] = jnp.dot(hg, wgm_ref[...],
                         preferred_element_type=jnp.float32) + bgm_ref[...]
    out_ref[3] = jax.nn.softplus(
        jnp.dot(hg, wgs_ref[...], preferred_element_type=jnp.float32)
        + bgs_ref[...])


def _tc_heads(agg2, deg_in_parts, gcn_W, gcn_b,
              wamT, bam, wasT, bas, wgmT, bgm, wgsT, bgs):
    return pl.pallas_call(
        _tc_heads_body,
        grid=(GRID2,),
        in_specs=[
            pl.BlockSpec((2, NB2, H), lambda i: (0, i, 0)),
            pl.BlockSpec((NTILES, NB2), lambda i: (0, i)),
            pl.BlockSpec((H, H), lambda i: (0, 0)),
            pl.BlockSpec((1, H), lambda i: (0, 0)),
            pl.BlockSpec((H, I), lambda i: (0, 0)),
            pl.BlockSpec((1, I), lambda i: (0, 0)),
            pl.BlockSpec((H, I), lambda i: (0, 0)),
            pl.BlockSpec((1, I), lambda i: (0, 0)),
            pl.BlockSpec((H, I), lambda i: (0, 0)),
            pl.BlockSpec((1, I), lambda i: (0, 0)),
            pl.BlockSpec((H, I), lambda i: (0, 0)),
            pl.BlockSpec((1, I), lambda i: (0, 0)),
        ],
        out_specs=pl.BlockSpec((4, NB2, I), lambda i: (0, i, 0)),
        out_shape=jax.ShapeDtypeStruct((4, N, I), jnp.float32),
    )(agg2, deg_in_parts, gcn_W, gcn_b, wamT, bam, wasT, bas, wgmT, bgm, wgsT, bgs)


def kernel(actions, hidden, edge_index, W_ih, W_hh, b_ih, b_hh, ln_g, ln_b,
           gcn_W, gcn_b, zIA_mu_W, zIA_mu_b, zIA_std_W, zIA_std_b,
           zIG_mu_W, zIG_mu_b, zIG_std_W, zIG_std_b):
    f32 = jnp.float32
    src = edge_index[0].reshape(NTILES, RPT_E)
    dst = edge_index[1].reshape(NTILES, RPT_E)
    ar = jnp.arange(TPAD, dtype=jnp.int32)
    pad_src = jnp.broadcast_to(N + ar[None, :], (NTILES, TPAD))
    ks = jnp.arange(NTILES, dtype=jnp.int32)
    pad_dst = (ks[:, None] * 997 + ar[None, :] * 131) % N
    src_t = jnp.concatenate([src, pad_src], axis=1)
    dst_gat = jnp.concatenate([dst, pad_dst], axis=1)

    dego_parts, degi_parts = _sc_degrees(edge_index)

    h_final, x_ln = _tc_gru(
        actions.reshape(N, T * A), hidden, W_ih.T, W_hh.T,
        b_ih.reshape(1, 3 * H), b_hh.reshape(1, 3 * H),
        ln_g.reshape(1, H), ln_b.reshape(1, H))
    x_scaled = _tc_scale(x_ln, dego_parts)

    zeros = jnp.zeros((RPT, H), f32)
    agg2 = _sc_gather_scatter(x_scaled,
                              src_t.reshape(NTILES, NG, GCH, CHUNK),
                              dst_gat.reshape(NTILES, NG, GCH, CHUNK),
                              zeros)

    heads = _tc_heads(
        agg2, degi_parts, gcn_W, gcn_b.reshape(1, H),
        zIA_mu_W.T, zIA_mu_b.reshape(1, I),
        zIA_std_W.T, zIA_std_b.reshape(1, I),
        zIG_mu_W.T, zIG_mu_b.reshape(1, I),
        zIG_std_W.T, zIG_std_b.reshape(1, I))

    return heads[2], heads[3], heads[0], heads[1], h_final[None]


# speedup vs baseline: 1.1177x; 1.1177x over previous
import functools

import jax
import jax.numpy as jnp
from jax import lax
from jax.experimental import pallas as pl
from jax.experimental.pallas import tpu as pltpu
from jax.experimental.pallas import tpu_sc as plsc

N = 10000
E = 320000
T = 8
A = 16
H = 128
I = 64

NTILES = 32
LANES = 16
CHUNK = 80
GCH = 16
NG = 8
CH = NG * GCH
RPT_E = E // NTILES
EPT = CH * CHUNK
TPAD = EPT - RPT_E
NP = 10240
RPT = 625
NAGG = 16 * RPT
NB = 640
GRIDA = NP // NB
NB2 = 1024
GRID2 = NP // NB2

_mesh = plsc.VectorSubcoreMesh(core_axis_name="c", subcore_axis_name="s")
_sc_params = pltpu.CompilerParams(needs_layout_passes=False,
                                  use_tc_tiling_on_sc=False)


@functools.partial(
    pl.kernel,
    out_type=(jax.ShapeDtypeStruct((NTILES, NP), jnp.float32),
              jax.ShapeDtypeStruct((NTILES, NP), jnp.float32)),
    mesh=_mesh,
    scratch_types=[
        pltpu.VMEM((RPT_E,), jnp.int32),
        pltpu.VMEM((RPT_E,), jnp.int32),
        pltpu.VMEM((NP,), jnp.float32),
        pltpu.VMEM((NP,), jnp.float32),
    ],
    compiler_params=_sc_params,
)
def _sc_degrees(edge_hbm, dego_hbm, degi_hbm, src_v, dst_v, ho_v, hi_v):
    c = lax.axis_index("c")
    s = lax.axis_index("s")
    wid = c * 16 + s
    pltpu.sync_copy(edge_hbm.at[0, pl.ds(wid * RPT_E, RPT_E)], src_v)
    pltpu.sync_copy(edge_hbm.at[1, pl.ds(wid * RPT_E, RPT_E)], dst_v)
    zeros = jnp.zeros((LANES,), jnp.float32)
    ones = jnp.ones((LANES,), jnp.float32)

    def zero_body(v, _):
        ho_v[pl.ds(v * LANES, LANES)] = zeros
        hi_v[pl.ds(v * LANES, LANES)] = zeros
        return _

    lax.fori_loop(0, NP // LANES, zero_body, 0)

    def hist_body(v, _):
        si = src_v[pl.ds(v * LANES, LANES)]
        di = dst_v[pl.ds(v * LANES, LANES)]
        plsc.addupdate_scatter(ho_v, [si], ones)
        plsc.addupdate_scatter(hi_v, [di], ones)
        return _

    lax.fori_loop(0, RPT_E // LANES, hist_body, 0)
    pltpu.sync_copy(ho_v, dego_hbm.at[wid])
    pltpu.sync_copy(hi_v, degi_hbm.at[wid])


def _tc_gru_body(a_ref, h0_ref, wih_ref, whh_ref, bih_ref, bhh_ref,
                 lng_ref, lnb_ref, degp_ref, hfin_ref, xsc_ref):
    blk = pl.program_id(0)
    wih = wih_ref[...]
    whh = whh_ref[...]
    bih = bih_ref[...]
    bhh = bhh_ref[...]
    a = a_ref[...]
    h = h0_ref[0]
    for t in range(T):
        x_t = a[:, t * A:(t + 1) * A]
        gi = jnp.dot(x_t, wih, preferred_element_type=jnp.float32) + bih
        gh = jnp.dot(h, whh, preferred_element_type=jnp.float32) + bhh
        i_r = gi[:, :H]
        i_z = gi[:, H:2 * H]
        i_n = gi[:, 2 * H:]
        h_r = gh[:, :H]
        h_z = gh[:, H:2 * H]
        h_n = gh[:, 2 * H:]
        r = jax.nn.sigmoid(i_r + h_r)
        z = jax.nn.sigmoid(i_z + h_z)
        n = jnp.tanh(i_n + r * h_n)
        h = (1.0 - z) * n + z * h
    hfin_ref[...] = h

    mu = jnp.mean(h, axis=1, keepdims=True)
    d = h - mu
    var = jnp.mean(d * d, axis=1, keepdims=True)
    x = d * lax.rsqrt(var + 1e-5) * lng_ref[...] + lnb_ref[...]

    deg = jnp.sum(degp_ref[...], axis=0).reshape(NB, 1)
    norm_src = lax.rsqrt(jnp.maximum(deg, 1.0))
    row = jax.lax.broadcasted_iota(jnp.int32, (NB, 1), 0) + blk * NB
    xsc_ref[...] = jnp.where(row < N, x * norm_src, 0.0)


def _tc_gru(acts, hidden, wihT, whhT, bih, bhh, lng, lnb, deg_out_parts):
    return pl.pallas_call(
        _tc_gru_body,
        grid=(GRIDA,),
        in_specs=[
            pl.BlockSpec((NB, T * A), lambda i: (i, 0)),
            pl.BlockSpec((1, NB, H), lambda i: (0, i, 0)),
            pl.BlockSpec((A, 3 * H), lambda i: (0, 0)),
            pl.BlockSpec((H, 3 * H), lambda i: (0, 0)),
            pl.BlockSpec((1, 3 * H), lambda i: (0, 0)),
            pl.BlockSpec((1, 3 * H), lambda i: (0, 0)),
            pl.BlockSpec((1, H), lambda i: (0, 0)),
            pl.BlockSpec((1, H), lambda i: (0, 0)),
            pl.BlockSpec((NTILES, NB), lambda i: (0, i)),
        ],
        out_specs=[
            pl.BlockSpec((NB, H), lambda i: (i, 0)),
            pl.BlockSpec((NB, H), lambda i: (i, 0)),
        ],
        out_shape=[
            jax.ShapeDtypeStruct((N, H), jnp.float32),
            jax.ShapeDtypeStruct((NP, H), jnp.float32),
        ],
    )(acts, hidden, wihT, whhT, bih, bhh, lng, lnb, deg_out_parts)


@functools.partial(
    pl.kernel,
    out_type=jax.ShapeDtypeStruct((2, NAGG, H), jnp.float32),
    mesh=_mesh,
    scratch_types=[
        pltpu.VMEM((2, GCH, CHUNK), jnp.int32),
        pltpu.VMEM((2, GCH, CHUNK), jnp.int32),
        pltpu.VMEM((4, CHUNK, H), jnp.float32),
        pltpu.VMEM_SHARED((NAGG, H), jnp.float32),
        pltpu.SemaphoreType.DMA,
        pltpu.SemaphoreType.DMA,
        pltpu.SemaphoreType.DMA,
        pltpu.SemaphoreType.DMA,
    ],
    compiler_params=_sc_params,
)
def _sc_gather_scatter(x_hbm, src_hbm, dst_hbm, zeros_hbm, out_hbm,
                       src_v, dst_v, bufs, agg_sh, g0, g1, s0, s1):
    c = lax.axis_index("c")
    s = lax.axis_index("s")
    wid = c * 16 + s
    gsems = (g0, g1)
    ssems = (s0, s1)

    def srow(ci):
        return src_v.at[(ci // GCH) % 2, ci % GCH]

    def drow(ci):
        return dst_v.at[(ci // GCH) % 2, ci % GCH]

    pltpu.sync_copy(zeros_hbm, agg_sh.at[pl.ds(s * RPT, RPT)])
    pltpu.sync_copy(src_hbm.at[wid, 0], src_v.at[0])
    pltpu.sync_copy(dst_hbm.at[wid, 0], dst_v.at[0])
    plsc.subcore_barrier()

    pltpu.async_copy(x_hbm.at[srow(0)], bufs.at[0], g0)
    pltpu.async_copy(x_hbm.at[srow(1)], bufs.at[1], g1)

    def group(g, _):
        slot = g % 2
        nslot = (g + 1) % 2
        for j in range(GCH):
            if j == 2:
                @pl.when(g < NG - 1)
                def _prefetch():
                    pltpu.sync_copy(src_hbm.at[wid, g + 1], src_v.at[nslot])
                    pltpu.sync_copy(dst_hbm.at[wid, g + 1], dst_v.at[nslot])
            pltpu.make_async_copy(x_hbm.at[src_v.at[slot, j]],
                                  bufs.at[j % 4], gsems[j % 2]).wait()
            if j >= 2:
                pltpu.make_async_copy(bufs.at[(j - 2) % 4],
                                      agg_sh.at[dst_v.at[slot, j - 2]],
                                      ssems[j % 2]).wait()
            else:
                @pl.when(g > 0)
                def _wait_prev():
                    pltpu.make_async_copy(bufs.at[(j - 2) % 4],
                                          agg_sh.at[dst_v.at[nslot,
                                                             GCH + j - 2]],
                                          ssems[j % 2]).wait()
            pltpu.async_copy(bufs.at[j % 4], agg_sh.at[dst_v.at[slot, j]],
                             ssems[j % 2], add=True)
            if j < GCH - 2:
                pltpu.async_copy(x_hbm.at[src_v.at[slot, j + 2]],
                                 bufs.at[(j + 2) % 4], gsems[j % 2])
            else:
                @pl.when(g < NG - 1)
                def _fire_next():
                    pltpu.async_copy(x_hbm.at[src_v.at[nslot, j + 2 - GCH]],
                                     bufs.at[(j + 2) % 4], gsems[j % 2])
        return _

    lax.fori_loop(0, NG, group, 0)

    pltpu.make_async_copy(bufs.at[(CH - 2) % 4], agg_sh.at[drow(CH - 2)],
                          ssems[(CH - 2) % 2]).wait()
    pltpu.make_async_copy(bufs.at[(CH - 1) % 4], agg_sh.at[drow(CH - 1)],
                          ssems[(CH - 1) % 2]).wait()

    plsc.subcore_barrier()
    pltpu.sync_copy(agg_sh.at[pl.ds(s * RPT, RPT)],
                    out_hbm.at[c, pl.ds(s * RPT, RPT)])


def _tc_heads_body(agg2_ref, degp_ref, gw_ref, gb_ref,
                   wam_ref, bam_ref, was_ref, bas_ref,
                   wgm_ref, bgm_ref, wgs_ref, bgs_ref,
                   am_ref, as_ref, gm_ref, gs_ref):
    agg = agg2_ref[0] + agg2_ref[1]
    deg = jnp.sum(degp_ref[...], axis=0).reshape(NB2, 1)
    norm_dst = lax.rsqrt(jnp.maximum(deg, 1.0))
    hg = jnp.dot(agg * norm_dst, gw_ref[...],
                 preferred_element_type=jnp.float32) + gb_ref[...]
    am_ref[...] = jnp.dot(hg, wam_ref[...],
                          preferred_element_type=jnp.float32) + bam_ref[...]
    as_ref[...] = jax.nn.softplus(
        jnp.dot(hg, was_ref[...], preferred_element_type=jnp.float32)
        + bas_ref[...])
    gm_ref[...] = jnp.dot(hg, wgm_ref[...],
                          preferred_element_type=jnp.float32) + bgm_ref[...]
    gs_ref[...] = jax.nn.softplus(
        jnp.dot(hg, wgs_ref[...], preferred_element_type=jnp.float32)
        + bgs_ref[...])


def _tc_heads(agg2, deg_in_parts, gcn_W, gcn_b,
              wamT, bam, wasT, bas, wgmT, bgm, wgsT, bgs):
    return pl.pallas_call(
        _tc_heads_body,
        grid=(GRID2,),
        in_specs=[
            pl.BlockSpec((2, NB2, H), lambda i: (0, i, 0)),
            pl.BlockSpec((NTILES, NB2), lambda i: (0, i)),
            pl.BlockSpec((H, H), lambda i: (0, 0)),
            pl.BlockSpec((1, H), lambda i: (0, 0)),
            pl.BlockSpec((H, I), lambda i: (0, 0)),
            pl.BlockSpec((1, I), lambda i: (0, 0)),
            pl.BlockSpec((H, I), lambda i: (0, 0)),
            pl.BlockSpec((1, I), lambda i: (0, 0)),
            pl.BlockSpec((H, I), lambda i: (0, 0)),
            pl.BlockSpec((1, I), lambda i: (0, 0)),
            pl.BlockSpec((H, I), lambda i: (0, 0)),
            pl.BlockSpec((1, I), lambda i: (0, 0)),
        ],
        out_specs=[pl.BlockSpec((NB2, I), lambda i: (i, 0))] * 4,
        out_shape=[jax.ShapeDtypeStruct((N, I), jnp.float32)] * 4,
    )(agg2, deg_in_parts, gcn_W, gcn_b, wamT, bam, wasT, bas, wgmT, bgm, wgsT, bgs)


def kernel(actions, hidden, edge_index, W_ih, W_hh, b_ih, b_hh, ln_g, ln_b,
           gcn_W, gcn_b, zIA_mu_W, zIA_mu_b, zIA_std_W, zIA_std_b,
           zIG_mu_W, zIG_mu_b, zIG_std_W, zIG_std_b):
    f32 = jnp.float32
    src = edge_index[0].reshape(NTILES, RPT_E)
    dst = edge_index[1].reshape(NTILES, RPT_E)
    ar = jnp.arange(TPAD, dtype=jnp.int32)
    pad_src = jnp.broadcast_to(N + ar[None, :], (NTILES, TPAD))
    ks = jnp.arange(NTILES, dtype=jnp.int32)
    pad_dst = (ks[:, None] * 997 + ar[None, :] * 131) % N
    src_t = jnp.concatenate([src, pad_src], axis=1)
    dst_gat = jnp.concatenate([dst, pad_dst], axis=1)

    dego_parts, degi_parts = _sc_degrees(edge_index)

    h_final, x_scaled = _tc_gru(
        actions.reshape(N, T * A), hidden, W_ih.T, W_hh.T,
        b_ih.reshape(1, 3 * H), b_hh.reshape(1, 3 * H),
        ln_g.reshape(1, H), ln_b.reshape(1, H), dego_parts)

    zeros = jnp.zeros((RPT, H), f32)
    agg2 = _sc_gather_scatter(x_scaled,
                              src_t.reshape(NTILES, NG, GCH, CHUNK),
                              dst_gat.reshape(NTILES, NG, GCH, CHUNK),
                              zeros)

    zIA_mu, zIA_std, zIG_mu, zIG_std = _tc_heads(
        agg2, degi_parts, gcn_W, gcn_b.reshape(1, H),
        zIA_mu_W.T, zIA_mu_b.reshape(1, I),
        zIA_std_W.T, zIA_std_b.reshape(1, I),
        zIG_mu_W.T, zIG_mu_b.reshape(1, I),
        zIG_std_W.T, zIG_std_b.reshape(1, I))

    return zIG_mu, zIG_std, zIA_mu, zIA_std, h_final[None]
